# initial kernel scaffold (unmeasured)
import jax
import jax.numpy as jnp
from jax import lax
from jax.experimental import pallas as pl
from jax.experimental.pallas import tpu as pltpu

B = 4
S = 1024
K = 16 * 128
N = 4096
HALF = 512
SB = 256
NSB = HALF // SB
NCH = B * NSB


def kernel(O, Wo):
    O2 = O.reshape(B, S, K)

    def body(o_hbm, wo_ref, out_hbm, o_stage, send_buf, recv_buf, my_buf,
             load_sem, store_sem, send_sem, recv_sem, credit_sem):
        my_z = lax.axis_index("z")
        peer = (lax.axis_index("x"), lax.axis_index("y"), 1 - my_z)
        my_s0 = my_z * HALF
        peer_s0 = (1 - my_z) * HALF

        barrier = pltpu.get_barrier_semaphore()
        pl.semaphore_signal(barrier, inc=1, device_id=peer,
                            device_id_type=pl.DeviceIdType.MESH)
        pl.semaphore_wait(barrier, 1)

        prev_rdma = None
        for k in range(NCH):
            b, sb = divmod(k, NSB)

            ld = pltpu.make_async_copy(
                o_hbm.at[b, pl.ds(peer_s0 + sb * SB, SB), :],
                o_stage.at[0], load_sem)
            ld.start()
            ld.wait()
            if prev_rdma is not None:
                prev_rdma.wait_send()
            send_buf[...] = jnp.dot(o_stage[0], wo_ref[...],
                                    preferred_element_type=jnp.float32)
            if k > 0:
                pl.semaphore_wait(credit_sem, 1)
            rdma = pltpu.make_async_remote_copy(
                src_ref=send_buf, dst_ref=recv_buf,
                send_sem=send_sem, recv_sem=recv_sem,
                device_id=peer, device_id_type=pl.DeviceIdType.MESH)
            rdma.start()
            prev_rdma = rdma

            ld2 = pltpu.make_async_copy(
                o_hbm.at[b, pl.ds(my_s0 + sb * SB, SB), :],
                o_stage.at[1], load_sem)
            ld2.start()
            ld2.wait()
            my_buf[...] = jnp.dot(o_stage[1], wo_ref[...],
                                  preferred_element_type=jnp.float32)

            rdma.wait_recv()
            my_buf[...] = my_buf[...] + recv_buf[...]
            if k < NCH - 1:
                pl.semaphore_signal(credit_sem, inc=1, device_id=peer,
                                    device_id_type=pl.DeviceIdType.MESH)
            st = pltpu.make_async_copy(
                my_buf, out_hbm.at[b, pl.ds(sb * SB, SB), :], store_sem)
            st.start()
            st.wait()

        prev_rdma.wait_send()

    return pl.pallas_call(
        body,
        out_shape=jax.ShapeDtypeStruct((B, HALF, N), jnp.float32),
        in_specs=[
            pl.BlockSpec(memory_space=pltpu.ANY),
            pl.BlockSpec(memory_space=pltpu.VMEM),
        ],
        out_specs=pl.BlockSpec(memory_space=pltpu.ANY),
        scratch_shapes=[
            pltpu.VMEM((2, SB, K), jnp.float32),
            pltpu.VMEM((SB, N), jnp.float32),
            pltpu.VMEM((SB, N), jnp.float32),
            pltpu.VMEM((SB, N), jnp.float32),
            pltpu.SemaphoreType.DMA,
            pltpu.SemaphoreType.DMA,
            pltpu.SemaphoreType.DMA,
            pltpu.SemaphoreType.DMA,
            pltpu.SemaphoreType.REGULAR,
        ],
        compiler_params=pltpu.CompilerParams(
            collective_id=0,
            vmem_limit_bytes=64 * 1024 * 1024,
        ),
    )(O2, Wo)


# baseline (device time: 518088 ns/iter reference)
import jax
import jax.numpy as jnp
from jax import lax
from jax.experimental import pallas as pl
from jax.experimental.pallas import tpu as pltpu

B = 4
S = 1024
K = 16 * 128
N = 4096
HALF = 512
SB = 256
NSB = HALF // SB
NCH = B * NSB


def kernel(O, Wo):
    O2 = O.reshape(B, S, K)

    def body(o_hbm, wo_ref, out_hbm, o_stage, send_buf, recv_buf, my_buf,
             load_sem, store_sem, send_sem, recv_sem, credit_sem):
        my_z = lax.axis_index("z")
        peer = (lax.axis_index("x"), lax.axis_index("y"), 1 - my_z)
        my_s0 = my_z * HALF
        peer_s0 = (1 - my_z) * HALF

        barrier = pltpu.get_barrier_semaphore()
        pl.semaphore_signal(barrier, inc=1, device_id=peer,
                            device_id_type=pl.DeviceIdType.MESH)
        pl.semaphore_wait(barrier, 1)

        prev_rdma = None
        for k in range(NCH):
            b, sb = divmod(k, NSB)

            ld = pltpu.make_async_copy(
                o_hbm.at[b, pl.ds(peer_s0 + sb * SB, SB), :],
                o_stage.at[0], load_sem)
            ld.start()
            ld.wait()
            if prev_rdma is not None:
                prev_rdma.wait_send()
            send_buf[...] = jnp.dot(o_stage[0], wo_ref[...],
                                    preferred_element_type=jnp.float32)
            if k > 0:
                pl.semaphore_wait(credit_sem, 1)
            rdma = pltpu.make_async_remote_copy(
                src_ref=send_buf, dst_ref=recv_buf,
                send_sem=send_sem, recv_sem=recv_sem,
                device_id=peer, device_id_type=pl.DeviceIdType.MESH)
            rdma.start()
            prev_rdma = rdma

            ld2 = pltpu.make_async_copy(
                o_hbm.at[b, pl.ds(my_s0 + sb * SB, SB), :],
                o_stage.at[1], load_sem)
            ld2.start()
            ld2.wait()
            my_buf[...] = jnp.dot(o_stage[1], wo_ref[...],
                                  preferred_element_type=jnp.float32)

            rdma.wait_recv()
            my_buf[...] = my_buf[...] + recv_buf[...]
            if k < NCH - 1:
                pl.semaphore_signal(credit_sem, inc=1, device_id=peer,
                                    device_id_type=pl.DeviceIdType.MESH)
            st = pltpu.make_async_copy(
                my_buf, out_hbm.at[b, pl.ds(sb * SB, SB), :], store_sem)
            st.start()
            st.wait()

        prev_rdma.wait_send()

    return pl.pallas_call(
        body,
        out_shape=jax.ShapeDtypeStruct((B, HALF, N), jnp.float32),
        in_specs=[
            pl.BlockSpec(memory_space=pl.ANY),
            pl.BlockSpec(memory_space=pltpu.VMEM),
        ],
        out_specs=pl.BlockSpec(memory_space=pl.ANY),
        scratch_shapes=[
            pltpu.VMEM((2, SB, K), jnp.float32),
            pltpu.VMEM((SB, N), jnp.float32),
            pltpu.VMEM((SB, N), jnp.float32),
            pltpu.VMEM((SB, N), jnp.float32),
            pltpu.SemaphoreType.DMA,
            pltpu.SemaphoreType.DMA,
            pltpu.SemaphoreType.DMA,
            pltpu.SemaphoreType.DMA,
            pltpu.SemaphoreType.REGULAR,
        ],
        compiler_params=pltpu.CompilerParams(
            collective_id=0,
            vmem_limit_bytes=64 * 1024 * 1024,
        ),
    )(O2, Wo)


# device time: 510071 ns/iter; 1.0157x vs baseline; 1.0157x over previous
import jax
import jax.numpy as jnp
from jax import lax
from jax.experimental import pallas as pl
from jax.experimental.pallas import tpu as pltpu

B = 4
S = 1024
K = 16 * 128
N = 4096
HALF = 512
SB = 256
NSB = HALF // SB
NCH = B * NSB


def kernel(O, Wo):
    O2 = O.reshape(B, S, K).astype(jnp.bfloat16)
    Wo16 = Wo.astype(jnp.bfloat16)

    def body(o_hbm, wo_ref, out_hbm, o_stage, send_buf, recv_buf, my_buf,
             load_sems, store_sems, send_sems, recv_sems, credit_sem):
        my_z = lax.axis_index("z")
        peer = (lax.axis_index("x"), lax.axis_index("y"), 1 - my_z)
        my_s0 = my_z * HALF
        peer_s0 = (1 - my_z) * HALF

        barrier = pltpu.get_barrier_semaphore()
        pl.semaphore_signal(barrier, inc=1, device_id=peer,
                            device_id_type=pl.DeviceIdType.MESH)
        pl.semaphore_wait(barrier, 1)

        def exchange(slot):
            return pltpu.make_async_remote_copy(
                src_ref=send_buf.at[slot], dst_ref=recv_buf.at[slot],
                send_sem=send_sems.at[slot], recv_sem=recv_sems.at[slot],
                device_id=peer, device_id_type=pl.DeviceIdType.MESH)

        def store(slot, b, sb):
            return pltpu.make_async_copy(
                my_buf.at[slot], out_hbm.at[b, pl.ds(sb * SB, SB), :],
                store_sems.at[slot])

        def step(k, _):
            b = k // NSB
            sb = k % NSB
            slot = lax.rem(k, 2)

            ld = pltpu.make_async_copy(
                o_hbm.at[b, pl.ds(peer_s0 + sb * SB, SB), :],
                o_stage.at[0], load_sems.at[0])
            ld.start()
            ld2 = pltpu.make_async_copy(
                o_hbm.at[b, pl.ds(my_s0 + sb * SB, SB), :],
                o_stage.at[1], load_sems.at[1])
            ld2.start()
            ld.wait()

            @pl.when(k >= 2)
            def _():
                exchange(slot).wait_send()

            send_buf[slot] = jnp.dot(o_stage[0], wo_ref[...],
                                     preferred_element_type=jnp.float32)

            @pl.when(k >= 2)
            def _():
                pl.semaphore_wait(credit_sem, 1)
            exchange(slot).start()

            ld2.wait()

            @pl.when(k >= 2)
            def _():
                store(slot, (k - 2) // NSB, (k - 2) % NSB).wait()

            my_buf[slot] = jnp.dot(o_stage[1], wo_ref[...],
                                   preferred_element_type=jnp.float32)

            exchange(slot).wait_recv()
            my_buf[slot] = my_buf[slot] + recv_buf[slot]

            @pl.when(k <= NCH - 3)
            def _():
                pl.semaphore_signal(credit_sem, inc=1, device_id=peer,
                                    device_id_type=pl.DeviceIdType.MESH)
            store(slot, b, sb).start()
            return 0

        lax.fori_loop(0, NCH, step, 0)

        for k in (NCH - 2, NCH - 1):
            exchange(k % 2).wait_send()
            store(k % 2, k // NSB, k % NSB).wait()

    return pl.pallas_call(
        body,
        out_shape=jax.ShapeDtypeStruct((B, HALF, N), jnp.float32),
        in_specs=[
            pl.BlockSpec(memory_space=pl.ANY),
            pl.BlockSpec(memory_space=pltpu.VMEM),
        ],
        out_specs=pl.BlockSpec(memory_space=pl.ANY),
        scratch_shapes=[
            pltpu.VMEM((2, SB, K), jnp.bfloat16),
            pltpu.VMEM((2, SB, N), jnp.float32),
            pltpu.VMEM((2, SB, N), jnp.float32),
            pltpu.VMEM((2, SB, N), jnp.float32),
            pltpu.SemaphoreType.DMA((2,)),
            pltpu.SemaphoreType.DMA((2,)),
            pltpu.SemaphoreType.DMA((2,)),
            pltpu.SemaphoreType.DMA((2,)),
            pltpu.SemaphoreType.REGULAR,
        ],
        compiler_params=pltpu.CompilerParams(
            collective_id=0,
            vmem_limit_bytes=64 * 1024 * 1024,
        ),
    )(O2, Wo16)


# device time: 447447 ns/iter; 1.1579x vs baseline; 1.1400x over previous
import jax
import jax.numpy as jnp
from jax import lax
from jax.experimental import pallas as pl
from jax.experimental.pallas import tpu as pltpu

B = 4
S = 1024
K = 16 * 128
N = 4096
HALF = 512
SB = 256
NSB = HALF // SB
NCH = B * NSB


def kernel(O, Wo):
    O2 = O.reshape(B, S, K).astype(jnp.bfloat16)
    Wo16 = Wo.astype(jnp.bfloat16)

    def body(o_hbm, wo_ref, out_hbm, o_stage, send_buf, recv_buf, my_buf,
             load_sems, store_sems, send_sems, recv_sems, credit_sem):
        my_z = lax.axis_index("z")
        peer = (lax.axis_index("x"), lax.axis_index("y"), 1 - my_z)
        my_s0 = my_z * HALF
        peer_s0 = (1 - my_z) * HALF

        barrier = pltpu.get_barrier_semaphore()
        pl.semaphore_signal(barrier, inc=1, device_id=peer,
                            device_id_type=pl.DeviceIdType.MESH)
        pl.semaphore_wait(barrier, 1)

        def exchange(slot):
            return pltpu.make_async_remote_copy(
                src_ref=send_buf.at[slot], dst_ref=recv_buf.at[slot],
                send_sem=send_sems.at[slot], recv_sem=recv_sems.at[slot],
                device_id=peer, device_id_type=pl.DeviceIdType.MESH)

        def store(slot, b, sb):
            return pltpu.make_async_copy(
                my_buf.at[slot], out_hbm.at[b, pl.ds(sb * SB, SB), :],
                store_sems.at[slot])

        def step(k, _):
            b = k // NSB
            sb = k % NSB
            slot = lax.rem(k, 2)
            pslot = lax.rem(k + 1, 2)

            ld = pltpu.make_async_copy(
                o_hbm.at[b, pl.ds(peer_s0 + sb * SB, SB), :],
                o_stage.at[0], load_sems.at[0])
            ld.start()
            ld2 = pltpu.make_async_copy(
                o_hbm.at[b, pl.ds(my_s0 + sb * SB, SB), :],
                o_stage.at[1], load_sems.at[1])
            ld2.start()
            ld.wait()

            @pl.when(k >= 2)
            def _():
                exchange(slot).wait_send()

            send_buf[slot] = jnp.dot(o_stage[0], wo_ref[...],
                                     preferred_element_type=jnp.float32)

            @pl.when(k >= 2)
            def _():
                pl.semaphore_wait(credit_sem, 1)
            exchange(slot).start()

            ld2.wait()

            @pl.when(k >= 2)
            def _():
                store(slot, (k - 2) // NSB, (k - 2) % NSB).wait()

            my_buf[slot] = jnp.dot(o_stage[1], wo_ref[...],
                                   preferred_element_type=jnp.float32)

            @pl.when(k >= 1)
            def _():
                exchange(pslot).wait_recv()
                my_buf[pslot] = my_buf[pslot] + recv_buf[pslot]

            @pl.when(jnp.logical_and(k >= 1, k <= NCH - 2))
            def _():
                pl.semaphore_signal(credit_sem, inc=1, device_id=peer,
                                    device_id_type=pl.DeviceIdType.MESH)

            @pl.when(k >= 1)
            def _():
                store(pslot, (k - 1) // NSB, (k - 1) % NSB).start()
            return 0

        lax.fori_loop(0, NCH, step, 0)

        last = NCH - 1
        lslot = last % 2
        exchange(lslot).wait_recv()
        my_buf[lslot] = my_buf[lslot] + recv_buf[lslot]
        store(lslot, last // NSB, last % NSB).start()
        for k in (NCH - 2, NCH - 1):
            exchange(k % 2).wait_send()
            store(k % 2, k // NSB, k % NSB).wait()

    return pl.pallas_call(
        body,
        out_shape=jax.ShapeDtypeStruct((B, HALF, N), jnp.float32),
        in_specs=[
            pl.BlockSpec(memory_space=pl.ANY),
            pl.BlockSpec(memory_space=pltpu.VMEM),
        ],
        out_specs=pl.BlockSpec(memory_space=pl.ANY),
        scratch_shapes=[
            pltpu.VMEM((2, SB, K), jnp.bfloat16),
            pltpu.VMEM((2, SB, N), jnp.float32),
            pltpu.VMEM((2, SB, N), jnp.float32),
            pltpu.VMEM((2, SB, N), jnp.float32),
            pltpu.SemaphoreType.DMA((2,)),
            pltpu.SemaphoreType.DMA((2,)),
            pltpu.SemaphoreType.DMA((2,)),
            pltpu.SemaphoreType.DMA((2,)),
            pltpu.SemaphoreType.REGULAR,
        ],
        compiler_params=pltpu.CompilerParams(
            collective_id=0,
            vmem_limit_bytes=64 * 1024 * 1024,
        ),
    )(O2, Wo16)


# device time: 441465 ns/iter; 1.1736x vs baseline; 1.0136x over previous
import jax
import jax.numpy as jnp
from jax import lax
from jax.experimental import pallas as pl
from jax.experimental.pallas import tpu as pltpu

B = 4
S = 1024
K = 16 * 128
N = 4096
HALF = 512
SB = 256
NSB = HALF // SB
NCH = B * NSB


def kernel(O, Wo):
    O2 = O.reshape(B, S, K)

    def body(o_hbm, wo_ref, out_hbm, o_stage, send_buf, recv_buf, my_buf,
             load_sems, store_sems, send_sems, recv_sems, credit_sem):
        my_z = lax.axis_index("z")
        peer = (lax.axis_index("x"), lax.axis_index("y"), 1 - my_z)
        my_s0 = my_z * HALF
        peer_s0 = (1 - my_z) * HALF

        barrier = pltpu.get_barrier_semaphore()
        pl.semaphore_signal(barrier, inc=1, device_id=peer,
                            device_id_type=pl.DeviceIdType.MESH)
        pl.semaphore_wait(barrier, 1)

        def exchange(slot):
            return pltpu.make_async_remote_copy(
                src_ref=send_buf.at[slot], dst_ref=recv_buf.at[slot],
                send_sem=send_sems.at[slot], recv_sem=recv_sems.at[slot],
                device_id=peer, device_id_type=pl.DeviceIdType.MESH)

        def store(slot, b, sb):
            return pltpu.make_async_copy(
                my_buf.at[slot], out_hbm.at[b, pl.ds(sb * SB, SB), :],
                store_sems.at[slot])

        def step(k, _):
            b = k // NSB
            sb = k % NSB
            slot = lax.rem(k, 2)
            pslot = lax.rem(k + 1, 2)

            ld = pltpu.make_async_copy(
                o_hbm.at[b, pl.ds(peer_s0 + sb * SB, SB), :],
                o_stage.at[0], load_sems.at[0])
            ld.start()
            ld2 = pltpu.make_async_copy(
                o_hbm.at[b, pl.ds(my_s0 + sb * SB, SB), :],
                o_stage.at[1], load_sems.at[1])
            ld2.start()
            ld.wait()

            @pl.when(k >= 2)
            def _():
                exchange(slot).wait_send()

            send_buf[slot] = jnp.dot(o_stage[0], wo_ref[...],
                                     preferred_element_type=jnp.float32)

            @pl.when(k >= 2)
            def _():
                pl.semaphore_wait(credit_sem, 1)
            exchange(slot).start()

            ld2.wait()

            @pl.when(k >= 2)
            def _():
                store(slot, (k - 2) // NSB, (k - 2) % NSB).wait()

            my_buf[slot] = jnp.dot(o_stage[1], wo_ref[...],
                                   preferred_element_type=jnp.float32)

            @pl.when(k >= 1)
            def _():
                exchange(pslot).wait_recv()
                my_buf[pslot] = my_buf[pslot] + recv_buf[pslot]

            @pl.when(jnp.logical_and(k >= 1, k <= NCH - 2))
            def _():
                pl.semaphore_signal(credit_sem, inc=1, device_id=peer,
                                    device_id_type=pl.DeviceIdType.MESH)

            @pl.when(k >= 1)
            def _():
                store(pslot, (k - 1) // NSB, (k - 1) % NSB).start()
            return 0

        lax.fori_loop(0, NCH, step, 0)

        last = NCH - 1
        lslot = last % 2
        exchange(lslot).wait_recv()
        my_buf[lslot] = my_buf[lslot] + recv_buf[lslot]
        store(lslot, last // NSB, last % NSB).start()
        for k in (NCH - 2, NCH - 1):
            exchange(k % 2).wait_send()
            store(k % 2, k // NSB, k % NSB).wait()

    return pl.pallas_call(
        body,
        out_shape=jax.ShapeDtypeStruct((B, HALF, N), jnp.float32),
        in_specs=[
            pl.BlockSpec(memory_space=pl.ANY),
            pl.BlockSpec(memory_space=pltpu.VMEM),
        ],
        out_specs=pl.BlockSpec(memory_space=pl.ANY),
        scratch_shapes=[
            pltpu.VMEM((2, SB, K), jnp.float32),
            pltpu.VMEM((2, SB, N), jnp.float32),
            pltpu.VMEM((2, SB, N), jnp.float32),
            pltpu.VMEM((2, SB, N), jnp.float32),
            pltpu.SemaphoreType.DMA((2,)),
            pltpu.SemaphoreType.DMA((2,)),
            pltpu.SemaphoreType.DMA((2,)),
            pltpu.SemaphoreType.DMA((2,)),
            pltpu.SemaphoreType.REGULAR,
        ],
        compiler_params=pltpu.CompilerParams(
            collective_id=0,
            vmem_limit_bytes=64 * 1024 * 1024,
        ),
    )(O2, Wo)


# device time: 262366 ns/iter; 1.9747x vs baseline; 1.6826x over previous
import jax
import jax.numpy as jnp
from jax import lax
from jax.experimental import pallas as pl
from jax.experimental.pallas import tpu as pltpu

B = 4
S = 1024
K = 16 * 128
N = 4096
HALF = 512
SB = 256
NSB = HALF // SB
NCH = B * NSB
WB = 256
NWB = K // WB


def kernel(O, Wo):
    O2 = O.reshape(B, S, K)

    def body(o_hbm, wo_hbm, out_hbm, wo16, wo_stage, o_stage,
             send_buf, recv_buf, my_buf,
             wload_sems, load_sems, store_sems, send_sems, recv_sems,
             credit_sem):
        my_z = lax.axis_index("z")
        peer = (lax.axis_index("x"), lax.axis_index("y"), 1 - my_z)
        my_s0 = my_z * HALF
        peer_s0 = (1 - my_z) * HALF

        barrier = pltpu.get_barrier_semaphore()
        pl.semaphore_signal(barrier, inc=1, device_id=peer,
                            device_id_type=pl.DeviceIdType.MESH)
        pl.semaphore_wait(barrier, 1)

        def wo_load(r):
            return pltpu.make_async_copy(
                wo_hbm.at[pl.ds(r * WB, WB), :], wo_stage.at[r % 2],
                wload_sems.at[r % 2])

        wo_load(0).start()
        for r in range(NWB):
            if r + 1 < NWB:
                wo_load(r + 1).start()
            wo_load(r).wait()
            wo16[pl.ds(r * WB, WB), :] = wo_stage[r % 2].astype(jnp.bfloat16)

        def exchange(slot):
            return pltpu.make_async_remote_copy(
                src_ref=send_buf.at[slot], dst_ref=recv_buf.at[slot],
                send_sem=send_sems.at[slot], recv_sem=recv_sems.at[slot],
                device_id=peer, device_id_type=pl.DeviceIdType.MESH)

        def store(slot, b, sb):
            return pltpu.make_async_copy(
                my_buf.at[slot], out_hbm.at[b, pl.ds(sb * SB, SB), :],
                store_sems.at[slot])

        def step(k, _):
            b = k // NSB
            sb = k % NSB
            slot = lax.rem(k, 2)
            pslot = lax.rem(k + 1, 2)

            ld = pltpu.make_async_copy(
                o_hbm.at[b, pl.ds(peer_s0 + sb * SB, SB), :],
                o_stage.at[0], load_sems.at[0])
            ld.start()
            ld2 = pltpu.make_async_copy(
                o_hbm.at[b, pl.ds(my_s0 + sb * SB, SB), :],
                o_stage.at[1], load_sems.at[1])
            ld2.start()
            ld.wait()

            @pl.when(k >= 2)
            def _():
                exchange(slot).wait_send()

            send_buf[slot] = jnp.dot(
                o_stage[0].astype(jnp.bfloat16), wo16[...],
                preferred_element_type=jnp.float32).astype(jnp.bfloat16)

            @pl.when(k >= 2)
            def _():
                pl.semaphore_wait(credit_sem, 1)
            exchange(slot).start()

            ld2.wait()

            @pl.when(k >= 2)
            def _():
                store(slot, (k - 2) // NSB, (k - 2) % NSB).wait()

            my_buf[slot] = jnp.dot(
                o_stage[1].astype(jnp.bfloat16), wo16[...],
                preferred_element_type=jnp.float32)

            @pl.when(k >= 1)
            def _():
                exchange(pslot).wait_recv()
                my_buf[pslot] = (my_buf[pslot]
                                 + recv_buf[pslot].astype(jnp.float32))

            @pl.when(jnp.logical_and(k >= 1, k <= NCH - 2))
            def _():
                pl.semaphore_signal(credit_sem, inc=1, device_id=peer,
                                    device_id_type=pl.DeviceIdType.MESH)

            @pl.when(k >= 1)
            def _():
                store(pslot, (k - 1) // NSB, (k - 1) % NSB).start()
            return 0

        lax.fori_loop(0, NCH, step, 0)

        last = NCH - 1
        lslot = last % 2
        exchange(lslot).wait_recv()
        my_buf[lslot] = my_buf[lslot] + recv_buf[lslot].astype(jnp.float32)
        store(lslot, last // NSB, last % NSB).start()
        for k in (NCH - 2, NCH - 1):
            exchange(k % 2).wait_send()
            store(k % 2, k // NSB, k % NSB).wait()

    return pl.pallas_call(
        body,
        out_shape=jax.ShapeDtypeStruct((B, HALF, N), jnp.float32),
        in_specs=[
            pl.BlockSpec(memory_space=pl.ANY),
            pl.BlockSpec(memory_space=pl.ANY),
        ],
        out_specs=pl.BlockSpec(memory_space=pl.ANY),
        scratch_shapes=[
            pltpu.VMEM((K, N), jnp.bfloat16),
            pltpu.VMEM((2, WB, N), jnp.float32),
            pltpu.VMEM((2, SB, K), jnp.float32),
            pltpu.VMEM((2, SB, N), jnp.bfloat16),
            pltpu.VMEM((2, SB, N), jnp.bfloat16),
            pltpu.VMEM((2, SB, N), jnp.float32),
            pltpu.SemaphoreType.DMA((2,)),
            pltpu.SemaphoreType.DMA((2,)),
            pltpu.SemaphoreType.DMA((2,)),
            pltpu.SemaphoreType.DMA((2,)),
            pltpu.SemaphoreType.DMA((2,)),
            pltpu.SemaphoreType.REGULAR,
        ],
        compiler_params=pltpu.CompilerParams(
            collective_id=0,
            vmem_limit_bytes=64 * 1024 * 1024,
        ),
    )(O2, Wo)
